# baseline (speedup 1.0000x reference)
import functools

import jax
import jax.numpy as jnp
from jax import lax
from jax.experimental import pallas as pl
from jax.experimental.pallas import tpu as pltpu
from jax.experimental.pallas import tpu_sc as plsc

L = 16
NC = 2
NS = 16
NW = NC * NS
CHUNK = 256


def _transpose_kernel(vocab, dim, tt_hbm, tail_hbm, tr_hbm,
                      in_buf, out_buf, tail_v,
                      sem_in0, sem_in1, sem_out0, sem_out1):
    wid = lax.axis_index("s") * NC + lax.axis_index("c")
    n_chunks = vocab // 128
    per_w = (n_chunks + NW - 1) // NW
    sem_in = (sem_in0, sem_in1)
    sem_out = (sem_out0, sem_out1)

    @pl.when(wid == 0)
    def _copy_tail():
        pltpu.sync_copy(tail_hbm, tail_v)
        pltpu.sync_copy(tail_v, tr_hbm.at[pl.ds((vocab // 128) * 64, 32)])

    iota16 = lax.iota(jnp.int32, 16)
    fvecs = [iota16 + jnp.full((L,), 16 * t, jnp.int32) for t in range(4)]

    def in_copy(c, b):
        chunk = c * NW + wid
        return pltpu.make_async_copy(
            tt_hbm.at[:, pl.ds(chunk * 128, 128)], in_buf.at[b], sem_in[b])

    def out_copy(c, b):
        chunk = c * NW + wid
        return pltpu.make_async_copy(
            out_buf.at[b], tr_hbm.at[pl.ds(chunk * 64, 64)], sem_out[b])

    def started(c):
        return c * NW + wid < n_chunks

    def shuffle(b, q):
        for t in range(8):
            a = t // 4
            val = plsc.load_gather(
                in_buf.at[b],
                [fvecs[t % 4], jnp.full((L,), 2 * q + a, jnp.int32)])
            out_buf[b, q, pl.ds(16 * t, L)] = val

    @pl.when(started(0))
    def _prologue():
        in_copy(0, 0).start()

    def body(c, carry):
        for b in range(2):
            cc = 2 * c + b

            @pl.when(started(cc))
            def _process(cc=cc, b=b):
                in_copy(cc, b).wait()

                @pl.when(started(cc + 1))
                def _prefetch():
                    in_copy(cc + 1, 1 - b).start()

                @pl.when(cc >= 2)
                def _drain_out():
                    out_copy(cc - 2, b).wait()

                def q_body(q8, carry2):
                    for i in range(8):
                        shuffle(b, q8 * 8 + i)
                    return carry2

                lax.fori_loop(0, dim // 8, q_body, jnp.int32(0))
                out_copy(cc, b).start()
        return carry

    lax.fori_loop(0, (per_w + 1) // 2, body, jnp.int32(0))

    for b in range(2):
        @pl.when(started(b))
        def _drain_tail(b=b):
            pltpu.make_async_copy(
                tt_hbm.at[:, pl.ds(0, 128)], out_buf.at[b], sem_out[b]).wait()


def _cosine_gather_kernel(dim, bpw, u_idx_hbm, s_idx_hbm, tr_hbm, out_hbm,
                          u_idx_v, s_idx_v, uq_v, sq_v, ucb_v, scb_v,
                          u_buf, s_buf, out_v, sem_u, sem_s):
    wid = lax.axis_index("s") * NC + lax.axis_index("c")
    idx_rows = bpw // 128
    rows_per_chunk = CHUNK // 128
    base = wid * idx_rows

    pltpu.sync_copy(u_idx_hbm.at[pl.ds(base, idx_rows)], u_idx_v)
    pltpu.sync_copy(s_idx_hbm.at[pl.ds(base, idx_rows)], s_idx_v)

    for c in range(idx_rows):
        for l in range(128 // L):
            sl = pl.ds(l * L, L)
            for iv, qv, cbv in ((u_idx_v, uq_v, ucb_v), (s_idx_v, sq_v, scb_v)):
                r = iv[c, sl]
                row = jnp.bitwise_or(
                    lax.shift_left(lax.shift_right_logical(r, 7), 6),
                    jnp.bitwise_and(lax.shift_right_logical(r, 1),
                                    jnp.int32(63)))
                cb = lax.shift_left(jnp.bitwise_and(r, jnp.int32(1)), 6)
                qv[c, sl] = row
                cbv[pl.ds(c * 128 + l * L, L)] = cb

    iota16 = lax.iota(jnp.int32, 16)
    n_chunks = idx_rows // rows_per_chunk

    for chunk in range(n_chunks):
        copies = []
        for j in range(rows_per_chunk):
            jj = chunk * rows_per_chunk + j
            copies.append(pltpu.async_copy(
                tr_hbm.at[uq_v.at[jj]], u_buf.at[pl.ds(j * 128, 128)], sem_u))
            copies.append(pltpu.async_copy(
                tr_hbm.at[sq_v.at[jj]], s_buf.at[pl.ds(j * 128, 128)], sem_s))
        for cp in copies:
            cp.wait()

        def group_body(g, carry):
            j_loc = g * L + iota16
            cb_u = ucb_v[pl.ds(chunk * CHUNK + g * L, L)]
            cb_s = scb_v[pl.ds(chunk * CHUNK + g * L, L)]
            num = jnp.zeros((L,), jnp.float32)
            uu = jnp.zeros((L,), jnp.float32)
            ss = jnp.zeros((L,), jnp.float32)
            for k in range(dim):
                kv = jnp.full((L,), k, jnp.int32)
                u = plsc.load_gather(u_buf, [j_loc, cb_u + kv])
                s = plsc.load_gather(s_buf, [j_loc, cb_s + kv])
                num = num + u * s
                uu = uu + u * u
                ss = ss + s * s
            x = (jnp.maximum(uu, jnp.float32(1e-16))
                 * jnp.maximum(ss, jnp.float32(1e-16)))
            xi = lax.bitcast_convert_type(x, jnp.int32)
            yi = jnp.int32(0x5F3759DF) - lax.shift_right_arithmetic(xi, 1)
            y = lax.bitcast_convert_type(yi, jnp.float32)
            half_x = jnp.float32(0.5) * x
            for _ in range(3):
                y = y * (jnp.float32(1.5) - half_x * y * y)
            out_v[pl.ds(g * L, L)] = num * y
            return carry

        lax.fori_loop(0, CHUNK // L, group_body, jnp.int32(0))
        pltpu.sync_copy(
            out_v, out_hbm.at[pl.ds(wid * bpw + chunk * CHUNK, CHUNK)])


def kernel(user_emb, user_emb_offsets, section_emb, section_emb_offsets,
           node2vec_table):
    del user_emb_offsets, section_emb_offsets
    b = user_emb.shape[0]
    vocab, dim = node2vec_table.shape
    bpw = b // NW
    idx_rows = bpw // 128
    aligned = (vocab // 128) * 128
    tr_rows = (vocab // 128) * 64 + (vocab - aligned) // 2

    tt = jnp.transpose(node2vec_table)
    tail = jnp.reshape(
        lax.slice(node2vec_table, (aligned, 0), (vocab, dim)),
        ((vocab - aligned) // 2, 2 * dim))

    mesh = plsc.VectorSubcoreMesh(core_axis_name="c", subcore_axis_name="s")
    tr = pl.kernel(
        functools.partial(_transpose_kernel, vocab, dim),
        mesh=mesh,
        compiler_params=pltpu.CompilerParams(
            needs_layout_passes=False, use_tc_tiling_on_sc=True),
        out_type=jax.ShapeDtypeStruct((tr_rows, 2 * dim), jnp.float32),
        scratch_types=[
            pltpu.VMEM((2, dim, 128), jnp.float32),
            pltpu.VMEM((2, dim, 128), jnp.float32),
            pltpu.VMEM(((vocab - aligned) // 2, 2 * dim), jnp.float32),
            pltpu.SemaphoreType.DMA,
            pltpu.SemaphoreType.DMA,
            pltpu.SemaphoreType.DMA,
            pltpu.SemaphoreType.DMA,
        ],
    )(tt, tail)

    run = pl.kernel(
        functools.partial(_cosine_gather_kernel, dim, bpw),
        mesh=mesh,
        compiler_params=pltpu.CompilerParams(
            needs_layout_passes=False, use_tc_tiling_on_sc=False),
        out_type=jax.ShapeDtypeStruct((b,), jnp.float32),
        scratch_types=[
            pltpu.VMEM((idx_rows, 128), jnp.int32),
            pltpu.VMEM((idx_rows, 128), jnp.int32),
            pltpu.VMEM((idx_rows, 128), jnp.int32),
            pltpu.VMEM((idx_rows, 128), jnp.int32),
            pltpu.VMEM((idx_rows * 128,), jnp.int32),
            pltpu.VMEM((idx_rows * 128,), jnp.int32),
            pltpu.VMEM((CHUNK, 2 * dim), jnp.float32),
            pltpu.VMEM((CHUNK, 2 * dim), jnp.float32),
            pltpu.VMEM((CHUNK,), jnp.float32),
            pltpu.SemaphoreType.DMA,
            pltpu.SemaphoreType.DMA,
        ],
    )
    u_idx = user_emb.reshape(b // 128, 128).astype(jnp.int32)
    s_idx = section_emb.reshape(b // 128, 128).astype(jnp.int32)
    return run(u_idx, s_idx, tr)


# speedup vs baseline: 4.9888x; 4.9888x over previous
import functools

import jax
import jax.numpy as jnp
from jax import lax
from jax.experimental import pallas as pl
from jax.experimental.pallas import tpu as pltpu
from jax.experimental.pallas import tpu_sc as plsc

L = 16
NC = 2
NS = 16
NW = NC * NS
W = 8192
CHUNK = 256


def _transpose_body(dim, tt_ref, out_ref):
    a = tt_ref[...]
    out_ref[:, 0:dim] = jnp.transpose(a[:, 0:W // 2], (1, 0))
    out_ref[:, dim:2 * dim] = jnp.transpose(a[:, W // 2:W], (1, 0))


def _cosine_gather_kernel(dim, bpw, u_idx_hbm, s_idx_hbm, tr_hbm, out_hbm,
                          u_idx_v, s_idx_v, uq_v, sq_v, ucb_v, scb_v,
                          u_buf, s_buf, out_v, sem_u, sem_s):
    wid = lax.axis_index("s") * NC + lax.axis_index("c")
    idx_rows = bpw // 128
    rows_per_chunk = CHUNK // 128
    base = wid * idx_rows

    pltpu.sync_copy(u_idx_hbm.at[pl.ds(base, idx_rows)], u_idx_v)
    pltpu.sync_copy(s_idx_hbm.at[pl.ds(base, idx_rows)], s_idx_v)

    half_w = W // 2
    for c in range(idx_rows):
        for l in range(128 // L):
            sl = pl.ds(l * L, L)
            for iv, qv, cbv in ((u_idx_v, uq_v, ucb_v), (s_idx_v, sq_v, scb_v)):
                r = iv[c, sl]
                g_blk = lax.shift_right_logical(r, 12)
                o = jnp.bitwise_and(r, jnp.int32(W - 1))
                row = jnp.bitwise_or(
                    lax.shift_left(g_blk, 11),
                    jnp.bitwise_and(o, jnp.int32(half_w - 1)))
                cb = lax.shift_left(
                    lax.shift_right_logical(o, 11), 6)
                qv[c, sl] = row
                cbv[pl.ds(c * 128 + l * L, L)] = cb

    iota16 = lax.iota(jnp.int32, 16)
    n_chunks = idx_rows // rows_per_chunk

    for chunk in range(n_chunks):
        copies = []
        for j in range(rows_per_chunk):
            jj = chunk * rows_per_chunk + j
            copies.append(pltpu.async_copy(
                tr_hbm.at[uq_v.at[jj]], u_buf.at[pl.ds(j * 128, 128)], sem_u))
            copies.append(pltpu.async_copy(
                tr_hbm.at[sq_v.at[jj]], s_buf.at[pl.ds(j * 128, 128)], sem_s))
        for cp in copies:
            cp.wait()

        def group_body(g, carry):
            j_loc = g * L + iota16
            cb_u = ucb_v[pl.ds(chunk * CHUNK + g * L, L)]
            cb_s = scb_v[pl.ds(chunk * CHUNK + g * L, L)]
            num = jnp.zeros((L,), jnp.float32)
            uu = jnp.zeros((L,), jnp.float32)
            ss = jnp.zeros((L,), jnp.float32)
            for k in range(dim):
                kv = jnp.full((L,), k, jnp.int32)
                u = plsc.load_gather(u_buf, [j_loc, cb_u + kv])
                s = plsc.load_gather(s_buf, [j_loc, cb_s + kv])
                num = num + u * s
                uu = uu + u * u
                ss = ss + s * s
            x = (jnp.maximum(uu, jnp.float32(1e-16))
                 * jnp.maximum(ss, jnp.float32(1e-16)))
            xi = lax.bitcast_convert_type(x, jnp.int32)
            yi = jnp.int32(0x5F3759DF) - lax.shift_right_arithmetic(xi, 1)
            y = lax.bitcast_convert_type(yi, jnp.float32)
            half_x = jnp.float32(0.5) * x
            for _ in range(3):
                y = y * (jnp.float32(1.5) - half_x * y * y)
            out_v[pl.ds(g * L, L)] = num * y
            return carry

        lax.fori_loop(0, CHUNK // L, group_body, jnp.int32(0))
        pltpu.sync_copy(
            out_v, out_hbm.at[pl.ds(wid * bpw + chunk * CHUNK, CHUNK)])


def kernel(user_emb, user_emb_offsets, section_emb, section_emb_offsets,
           node2vec_table):
    del user_emb_offsets, section_emb_offsets
    b = user_emb.shape[0]
    vocab, dim = node2vec_table.shape
    bpw = b // NW
    idx_rows = bpw // 128
    grid = (vocab + W - 1) // W
    out_rows = grid * (W // 2)

    tt = jnp.transpose(node2vec_table)
    tr = pl.pallas_call(
        functools.partial(_transpose_body, dim),
        grid=(grid,),
        in_specs=[pl.BlockSpec((dim, W), lambda g: (0, g))],
        out_specs=pl.BlockSpec((W // 2, 2 * dim), lambda g: (g, 0)),
        out_shape=jax.ShapeDtypeStruct((out_rows, 2 * dim), jnp.float32),
    )(tt)

    mesh = plsc.VectorSubcoreMesh(core_axis_name="c", subcore_axis_name="s")
    run = pl.kernel(
        functools.partial(_cosine_gather_kernel, dim, bpw),
        mesh=mesh,
        compiler_params=pltpu.CompilerParams(
            needs_layout_passes=False, use_tc_tiling_on_sc=False),
        out_type=jax.ShapeDtypeStruct((b,), jnp.float32),
        scratch_types=[
            pltpu.VMEM((idx_rows, 128), jnp.int32),
            pltpu.VMEM((idx_rows, 128), jnp.int32),
            pltpu.VMEM((idx_rows, 128), jnp.int32),
            pltpu.VMEM((idx_rows, 128), jnp.int32),
            pltpu.VMEM((idx_rows * 128,), jnp.int32),
            pltpu.VMEM((idx_rows * 128,), jnp.int32),
            pltpu.VMEM((CHUNK, 2 * dim), jnp.float32),
            pltpu.VMEM((CHUNK, 2 * dim), jnp.float32),
            pltpu.VMEM((CHUNK,), jnp.float32),
            pltpu.SemaphoreType.DMA,
            pltpu.SemaphoreType.DMA,
        ],
    )
    u_idx = user_emb.reshape(b // 128, 128).astype(jnp.int32)
    s_idx = section_emb.reshape(b // 128, 128).astype(jnp.int32)
    return run(u_idx, s_idx, tr)
